# baseline (speedup 1.0000x reference)
import jax
import jax.numpy as jnp
from jax.experimental import pallas as pl

E = 32
BATCH = 640000
BLOCK = 6400


def _tc_body(x0_ref, x1_ref, out_ref):
    x0 = x0_ref[...]
    x1 = x1_ref[...]
    ar = x0[:, :E]
    ai = x0[:, E:]
    br = x1[:, :E]
    bi = x1[:, E:]
    out_ref[...] = jnp.concatenate([ar * br - ai * bi, ai * br + ar * bi],
                                   axis=1)


def kernel(in0, in1):
    n = in0.shape[0]
    grid = (n // BLOCK,)
    return pl.pallas_call(
        _tc_body,
        grid=grid,
        in_specs=[
            pl.BlockSpec((BLOCK, 2 * E), lambda i: (i, 0)),
            pl.BlockSpec((BLOCK, 2 * E), lambda i: (i, 0)),
        ],
        out_specs=pl.BlockSpec((BLOCK, 2 * E), lambda i: (i, 0)),
        out_shape=jax.ShapeDtypeStruct((n, 2 * E), jnp.float32),
    )(in0, in1)
